# baseline (speedup 1.0000x reference)
import math
from functools import partial

import jax
import jax.numpy as jnp
from jax.experimental import pallas as pl

N_NODES = 10000
N_GRAPHS = 128
NUM_EMB = 64
NUM_HEADS = 2


def _leaky(x):
    return jnp.where(x >= 0, x, 0.01 * x)


def _mlp3_body(a_ref, w1, b1, w2, b2, w3, b3, o_ref):
    a = a_ref[...]
    h = _leaky(jnp.dot(a, w1[...], preferred_element_type=jnp.float32) + b1[...])
    h = _leaky(jnp.dot(h, w2[...], preferred_element_type=jnp.float32) + b2[...])
    h = jnp.dot(h, w3[...], preferred_element_type=jnp.float32) + b3[...]
    o_ref[...] = h


def _mlp3(a, p, bm=2048):
    M, F = a.shape
    D = p["Ws"][2].shape[1]
    grid = (pl.cdiv(M, bm),)
    full = lambda shape: pl.BlockSpec(shape, lambda i: (0,) * len(shape))
    return pl.pallas_call(
        _mlp3_body,
        grid=grid,
        in_specs=[
            pl.BlockSpec((bm, F), lambda i: (i, 0)),
            full(p["Ws"][0].shape), full((1, p["bs"][0].shape[0])),
            full(p["Ws"][1].shape), full((1, p["bs"][1].shape[0])),
            full(p["Ws"][2].shape), full((1, p["bs"][2].shape[0])),
        ],
        out_specs=pl.BlockSpec((bm, D), lambda i: (i, 0)),
        out_shape=jax.ShapeDtypeStruct((M, D), jnp.float32),
    )(a, p["Ws"][0], p["bs"][0][None], p["Ws"][1], p["bs"][1][None],
      p["Ws"][2], p["bs"][2][None])


def _graph_layer_norm(x, batch, num_graphs, eps=1e-5):
    cnt = jax.ops.segment_sum(jnp.ones((x.shape[0],), jnp.float32), batch, num_graphs)
    norm = jnp.clip(cnt, 1.0, None) * x.shape[1]
    mean = jax.ops.segment_sum(x, batch, num_graphs).sum(axis=-1) / norm
    x = x - mean[batch][:, None]
    var = jax.ops.segment_sum(x * x, batch, num_graphs).sum(axis=-1) / norm
    return x / jnp.sqrt(var + eps)[batch][:, None]


def _mlp_apply(x, p):
    n = len(p["Ws"])
    for i in range(n):
        x = x @ p["Ws"][i] + p["bs"][i]
        if i < n - 1:
            x = _leaky(x)
    return x


def kernel(x, edge_index, edge_attr, batch, cond, params):
    N, G = x.shape[0], cond.shape[0]
    o = _mlp3(x, params["x2h"])
    e = _mlp3(edge_attr, params["e2h"])
    c = _mlp3(cond, params["c2h"], bm=128)

    u = jnp.arange(N, dtype=edge_index.dtype)
    v = batch.astype(edge_index.dtype) + N
    aug_src = jnp.concatenate([edge_index[0], u, v])
    aug_dst = jnp.concatenate([edge_index[1], v, u])
    e_p = jnp.zeros((2 * N, e.shape[1]), jnp.float32).at[:, 0].set(1.0)
    aug_e = jnp.concatenate([e, e_p], axis=0)
    n_total = N + G
    cnt = jax.ops.segment_sum(jnp.ones((aug_dst.shape[0],), jnp.float32), aug_dst, n_total)
    loop_attr = jax.ops.segment_sum(aug_e, aug_dst, n_total) / jnp.clip(cnt, 1.0, None)[:, None]
    sl = jnp.arange(n_total, dtype=edge_index.dtype)
    aug_src = jnp.concatenate([aug_src, sl])
    aug_dst = jnp.concatenate([aug_dst, sl])
    aug_e = jnp.concatenate([aug_e, loop_attr], axis=0)
    aug_batch = jnp.concatenate([batch, jnp.arange(G, dtype=batch.dtype)])
    o = jnp.concatenate([o, c], axis=0)

    H, C = NUM_HEADS, NUM_EMB
    for p in params["layers"]:
        msg = jax.nn.relu(o[aug_src] + aug_e) + 1e-7
        agg = jax.ops.segment_sum(msg, aug_dst, n_total)
        genout = (agg + o) @ p["gen_W"] + p["gen_b"]

        xc = jnp.concatenate([o, genout], axis=1)
        q = (xc @ p["Wq"] + p["bq"]).reshape(-1, H, C)
        k = (xc @ p["Wk"] + p["bk"]).reshape(-1, H, C)
        vv = (xc @ p["Wv"] + p["bv"]).reshape(-1, H, C)
        ee = (aug_e @ p["We"]).reshape(-1, H, C)
        k_j = k[aug_src] + ee
        v_j = vv[aug_src] + ee
        alpha = (q[aug_dst] * k_j).sum(axis=-1) / math.sqrt(C)
        amax = jax.ops.segment_max(alpha, aug_dst, n_total)
        amax = jnp.where(jnp.isfinite(amax), amax, 0.0)
        ae = jnp.exp(alpha - amax[aug_dst])
        asum = jax.ops.segment_sum(ae, aug_dst, n_total)
        alpha = ae / (asum[aug_dst] + 1e-16)
        out = jax.ops.segment_sum(v_j * alpha[..., None], aug_dst, n_total).reshape(-1, H * C)
        t = out + (xc @ p["Wsk"] + p["bsk"])

        o = _graph_layer_norm(o + (t @ p["lin_W"] + p["lin_b"]), aug_batch, G)
        o = _graph_layer_norm(o + _mlp_apply(o, p["ff"]), aug_batch, G)

    node_cnt = jax.ops.segment_sum(jnp.ones((N,), jnp.float32), batch, G)
    gmp = jax.ops.segment_sum(o[:N], batch, G) / jnp.clip(node_cnt, 1.0, None)[:, None]
    glob = jnp.concatenate([gmp, o[N:], c], axis=1)
    o_final = jnp.concatenate([o[:N], c[batch]], axis=1)
    return (o_final, glob)


# speedup vs baseline: 7.3564x; 7.3564x over previous
import math
from functools import partial

import jax
import jax.numpy as jnp
from jax import lax
from jax.experimental import pallas as pl
from jax.experimental.pallas import tpu as pltpu
from jax.experimental.pallas import tpu_sc as plsc

N_NODES = 10000
N_GRAPHS = 128
NUM_EMB = 64
NUM_HEADS = 2

NC, NS, LANES = 2, 16, 16
NW = NC * NS
NPAD = 10240
PAD_ROW = NPAD - 1
CHUNK = 512


ROWS_W = NPAD // NW


def _sc_mesh():
    return plsc.VectorSubcoreMesh(core_axis_name="c", subcore_axis_name="s")


def _zero_acc(acc, rows, width):
    z = jnp.zeros((LANES,), jnp.float32)

    @pl.loop(0, rows)
    def _(r):
        for f in range(width // LANES):
            acc[r, pl.ds(f * LANES, LANES)] = z


def _gen_msg_body(o_hbm, es_hbm, src_hbm, dst_hbm, eid_hbm, off_hbm, out_hbm,
                  sidx, eidx, didx_sm, off_sm, gbuf, ebuf, acc, sem, sem2):
    c = lax.axis_index("c")
    s = lax.axis_index("s")
    w = c * NS + s
    row0 = w * ROWS_W

    _zero_acc(acc, ROWS_W, 64)
    pltpu.sync_copy(off_hbm, off_sm)
    iota = lax.iota(jnp.int32, LANES)
    ovec = plsc.load_gather(off_sm, [w + iota])
    lo = ovec[0]
    hi = ovec[1]
    lo8 = (lo // 8) * 8
    nj = (hi - lo8 + 127) // 128

    @pl.loop(0, nj)
    def _(j):
        base = lo8 + j * 128
        pltpu.sync_copy(src_hbm.at[pl.ds(base, 128)], sidx)
        pltpu.sync_copy(eid_hbm.at[pl.ds(base, 128)], eidx)
        pltpu.sync_copy(dst_hbm.at[pl.ds(base, 128)], didx_sm)
        cp1 = pltpu.make_async_copy(o_hbm.at[sidx], gbuf, sem)
        cp2 = pltpu.make_async_copy(es_hbm.at[eidx], ebuf, sem2)
        cp1.start()
        cp2.start()
        cp1.wait()
        cp2.wait()

        @pl.loop(0, 8)
        def _(g):
            dvec = didx_sm[pl.ds(g * LANES, LANES)] - row0
            for i in range(LANES):
                r = g * LANES + i
                pos = base + r
                dloc = dvec[i]
                valid = jnp.logical_and(pos >= lo, pos < hi)
                m = jnp.broadcast_to(valid, (LANES,))
                rowv = jnp.broadcast_to(dloc, (LANES,))
                for f in range(4):
                    sl = pl.ds(f * LANES, LANES)
                    v = jnp.maximum(gbuf[r, sl] + ebuf[r, sl], 0.0) + 1e-7
                    plsc.addupdate_scatter(acc, [rowv, iota + f * LANES], v, mask=m)

    pltpu.sync_copy(acc, out_hbm.at[pl.ds(row0, ROWS_W)])


BC_CH = 64


def _attn_body(dtab, stab, es_hbm, src_hbm, dst_hbm, eid_hbm, off_hbm, out_hbm,
               sidx, didx, eidx, off_v, dgath, sgath, ebuf, acc, sem, sem2, sem3):
    c = lax.axis_index("c")
    s = lax.axis_index("s")
    w = c * NS + s
    row0 = w * ROWS_W

    _zero_acc(acc, ROWS_W, 272)
    pltpu.sync_copy(off_hbm, off_v)
    iota = lax.iota(jnp.int32, LANES)
    ovec = plsc.load_gather(off_v, [w + iota])
    lo = ovec[0]
    hi = ovec[1]
    lo8 = (lo // 8) * 8
    nj = (hi - lo8 + BC_CH - 1) // BC_CH

    @pl.loop(0, nj)
    def _(j):
        base = lo8 + j * BC_CH
        pltpu.sync_copy(src_hbm.at[pl.ds(base, BC_CH)], sidx)
        pltpu.sync_copy(eid_hbm.at[pl.ds(base, BC_CH)], eidx)
        pltpu.sync_copy(dst_hbm.at[pl.ds(base, BC_CH)], didx)
        cps = [pltpu.make_async_copy(dtab.at[didx], dgath, sem),
               pltpu.make_async_copy(stab.at[sidx], sgath, sem2),
               pltpu.make_async_copy(es_hbm.at[eidx], ebuf, sem3)]
        for cp in cps:
            cp.start()
        for cp in cps:
            cp.wait()

        @pl.loop(0, BC_CH // LANES)
        def _(g):
            dvec = didx[pl.ds(g * LANES, LANES)] - row0
            for i in range(LANES):
                r = g * LANES + i
                pos = base + r
                valid = jnp.logical_and(pos >= lo, pos < hi)
                m = jnp.broadcast_to(valid, (LANES,))
                rowv = jnp.broadcast_to(dvec[i], (LANES,))
                qk = [dgath[r, pl.ds(f * LANES, LANES)] for f in range(8)]
                qe = [dgath[r, pl.ds(128 + f * LANES, LANES)] for f in range(8)]
                kv = [sgath[r, pl.ds(f * LANES, LANES)] for f in range(8)]
                vv = [sgath[r, pl.ds(128 + f * LANES, LANES)] for f in range(8)]
                ev = [ebuf[r, pl.ds(f * LANES, LANES)] for f in range(4)]
                svec = dgath[r, pl.ds(256, LANES)]
                m0 = qk[0] * kv[0]
                m1 = qk[4] * kv[4]
                for f in range(1, 4):
                    m0 = m0 + qk[f] * kv[f]
                    m1 = m1 + qk[4 + f] * kv[4 + f]
                for f in range(4):
                    m0 = m0 + qe[f] * ev[f]
                    m1 = m1 + qe[4 + f] * ev[f]
                a0 = jnp.sum(m0) * 0.125 - svec[0]
                a1 = jnp.sum(m1) * 0.125 - svec[1]
                zv = jnp.exp(jnp.where(iota == 0, a0, jnp.where(iota == 1, a1, 0.0)))
                zb0 = jnp.broadcast_to(zv[0], (LANES,))
                zb1 = jnp.broadcast_to(zv[1], (LANES,))
                for f in range(4):
                    sl = iota + f * LANES
                    plsc.addupdate_scatter(acc, [rowv, sl], vv[f] * zb0, mask=m)
                    plsc.addupdate_scatter(acc, [rowv, sl + 64], vv[4 + f] * zb1, mask=m)
                    plsc.addupdate_scatter(acc, [rowv, sl + 128], ev[f] * zb0, mask=m)
                    plsc.addupdate_scatter(acc, [rowv, sl + 192], ev[f] * zb1, mask=m)
                zrow = jnp.where(iota == 0, zb0, jnp.where(iota == 1, zb1, 0.0))
                plsc.addupdate_scatter(acc, [rowv, iota + 256], zrow, mask=m)

    pltpu.sync_copy(acc, out_hbm.at[pl.ds(row0, ROWS_W)])


def _sc_params():
    return pltpu.CompilerParams(use_tc_tiling_on_sc=False,
                                needs_layout_passes=False)


@jax.jit
def _sc_gen_msg(o_pad, e_store, src_s, dst_s, eid_s, offs):
    k = pl.kernel(
        _gen_msg_body,
        out_type=jax.ShapeDtypeStruct((NPAD, 64), jnp.float32),
        mesh=_sc_mesh(),
        compiler_params=_sc_params(),
        scratch_types=[
            pltpu.VMEM((128,), jnp.int32),
            pltpu.VMEM((128,), jnp.int32),
            pltpu.VMEM((128,), jnp.int32),
            pltpu.VMEM((48,), jnp.int32),
            pltpu.VMEM((128, 64), jnp.float32),
            pltpu.VMEM((128, 64), jnp.float32),
            pltpu.VMEM((ROWS_W, 64), jnp.float32),
            pltpu.SemaphoreType.DMA,
            pltpu.SemaphoreType.DMA,
        ],
    )
    return k(o_pad, e_store, src_s, dst_s, eid_s, offs)


@jax.jit
def _sc_attn(dtab, stab, e_store, src_s, dst_s, eid_s, offs):
    k = pl.kernel(
        _attn_body,
        out_type=jax.ShapeDtypeStruct((NPAD, 272), jnp.float32),
        mesh=_sc_mesh(),
        compiler_params=_sc_params(),
        scratch_types=[
            pltpu.VMEM((BC_CH,), jnp.int32),
            pltpu.VMEM((BC_CH,), jnp.int32),
            pltpu.VMEM((BC_CH,), jnp.int32),
            pltpu.VMEM((48,), jnp.int32),
            pltpu.VMEM((BC_CH, 272), jnp.float32),
            pltpu.VMEM((BC_CH, 256), jnp.float32),
            pltpu.VMEM((BC_CH, 64), jnp.float32),
            pltpu.VMEM((ROWS_W, 272), jnp.float32),
            pltpu.SemaphoreType.DMA,
            pltpu.SemaphoreType.DMA,
            pltpu.SemaphoreType.DMA,
        ],
    )
    return k(dtab, stab, e_store, src_s, dst_s, eid_s, offs)


def _lattr_body(es_hbm, dst_hbm, eid_hbm, off_hbm, out_hbm,
                didx, eidx, off_v, ebuf, acc, sem):
    c = lax.axis_index("c")
    s = lax.axis_index("s")
    w = c * NS + s
    row0 = w * ROWS_W

    _zero_acc(acc, ROWS_W, 80)
    pltpu.sync_copy(off_hbm, off_v)
    iota = lax.iota(jnp.int32, LANES)
    ovec = plsc.load_gather(off_v, [w + iota])
    lo = ovec[0]
    hi = ovec[1]
    lo8 = (lo // 8) * 8
    nj = (hi - lo8 + 127) // 128
    onerow = jnp.where(iota == 0, 1.0, 0.0)

    @pl.loop(0, nj)
    def _(j):
        base = lo8 + j * 128
        pltpu.sync_copy(eid_hbm.at[pl.ds(base, 128)], eidx)
        pltpu.sync_copy(dst_hbm.at[pl.ds(base, 128)], didx)
        cp = pltpu.make_async_copy(es_hbm.at[eidx], ebuf, sem)
        cp.start()
        cp.wait()

        @pl.loop(0, 8)
        def _(g):
            dvec = didx[pl.ds(g * LANES, LANES)] - row0
            for i in range(LANES):
                r = g * LANES + i
                pos = base + r
                valid = jnp.logical_and(pos >= lo, pos < hi)
                m = jnp.broadcast_to(valid, (LANES,))
                rowv = jnp.broadcast_to(dvec[i], (LANES,))
                for f in range(4):
                    plsc.addupdate_scatter(acc, [rowv, iota + f * LANES],
                                           ebuf[r, pl.ds(f * LANES, LANES)], mask=m)
                plsc.addupdate_scatter(acc, [rowv, iota + 64], onerow, mask=m)

    pltpu.sync_copy(acc, out_hbm.at[pl.ds(row0, ROWS_W)])


@jax.jit
def _sc_loop_attr(e_store, dst_s, eid_s, offs):
    k = pl.kernel(
        _lattr_body,
        out_type=jax.ShapeDtypeStruct((NPAD, 80), jnp.float32),
        mesh=_sc_mesh(),
        compiler_params=_sc_params(),
        scratch_types=[
            pltpu.VMEM((128,), jnp.int32),
            pltpu.VMEM((128,), jnp.int32),
            pltpu.VMEM((48,), jnp.int32),
            pltpu.VMEM((128, 64), jnp.float32),
            pltpu.VMEM((ROWS_W, 80), jnp.float32),
            pltpu.SemaphoreType.DMA,
        ],
    )
    return k(e_store, dst_s, eid_s, offs)


def _leaky(x):
    return jnp.where(x >= 0, x, 0.01 * x)


def _mlp3_body(a_ref, w1, b1, w2, b2, w3, b3, o_ref):
    a = a_ref[...]
    h = _leaky(jnp.dot(a, w1[...], preferred_element_type=jnp.float32) + b1[...])
    h = _leaky(jnp.dot(h, w2[...], preferred_element_type=jnp.float32) + b2[...])
    h = jnp.dot(h, w3[...], preferred_element_type=jnp.float32) + b3[...]
    o_ref[...] = h


def _mlp3(a, p, bm=2048):
    M, F = a.shape
    D = p["Ws"][2].shape[1]
    grid = (pl.cdiv(M, bm),)
    full = lambda shape: pl.BlockSpec(shape, lambda i: (0,) * len(shape))
    return pl.pallas_call(
        _mlp3_body,
        grid=grid,
        in_specs=[
            pl.BlockSpec((bm, F), lambda i: (i, 0)),
            full(p["Ws"][0].shape), full((1, p["bs"][0].shape[0])),
            full(p["Ws"][1].shape), full((1, p["bs"][1].shape[0])),
            full(p["Ws"][2].shape), full((1, p["bs"][2].shape[0])),
        ],
        out_specs=pl.BlockSpec((bm, D), lambda i: (i, 0)),
        out_shape=jax.ShapeDtypeStruct((M, D), jnp.float32),
    )(a, p["Ws"][0], p["bs"][0][None], p["Ws"][1], p["bs"][1][None],
      p["Ws"][2], p["bs"][2][None])


def _graph_layer_norm(x, batch, num_graphs, eps=1e-5):
    cnt = jax.ops.segment_sum(jnp.ones((x.shape[0],), jnp.float32), batch, num_graphs)
    norm = jnp.clip(cnt, 1.0, None) * x.shape[1]
    mean = jax.ops.segment_sum(x, batch, num_graphs).sum(axis=-1) / norm
    x = x - mean[batch][:, None]
    var = jax.ops.segment_sum(x * x, batch, num_graphs).sum(axis=-1) / norm
    return x / jnp.sqrt(var + eps)[batch][:, None]


def _mlp_apply(x, p):
    n = len(p["Ws"])
    for i in range(n):
        x = x @ p["Ws"][i] + p["bs"][i]
        if i < n - 1:
            x = _leaky(x)
    return x


def kernel(x, edge_index, edge_attr, batch, cond, params):
    N, G = x.shape[0], cond.shape[0]
    o = _mlp3(x, params["x2h"])
    e = _mlp3(edge_attr, params["e2h"])
    c = _mlp3(cond, params["c2h"], bm=128)

    u = jnp.arange(N, dtype=edge_index.dtype)
    v = batch.astype(edge_index.dtype) + N
    n_total = N + G
    NE = edge_index.shape[1]
    aug_batch = jnp.concatenate([batch, jnp.arange(G, dtype=batch.dtype)])
    o = jnp.concatenate([o, c], axis=0)

    def _sortset(srcs, dsts, eids):
        E = dsts.shape[0]
        E_pad = 128 * pl.cdiv(E, 128) + 128
        pad_e = E_pad - E
        perm = jnp.argsort(dsts)

        def _padi(a):
            return jnp.concatenate([a.astype(jnp.int32),
                                    jnp.zeros((pad_e,), jnp.int32)])

        dst_sorted = dsts[perm]
        offs = jnp.searchsorted(dst_sorted, jnp.arange(33, dtype=jnp.int32) * ROWS_W)
        offs = jnp.concatenate([offs.astype(jnp.int32), jnp.zeros((15,), jnp.int32)])
        src_p = _padi(srcs[perm]) if srcs is not None else None
        return src_p, _padi(dst_sorted), _padi(eids[perm]), offs

    src0 = jnp.concatenate([edge_index[0], u, v])
    dst0 = jnp.concatenate([edge_index[1], v, u])
    eid0 = jnp.concatenate([jnp.arange(NE, dtype=jnp.int32),
                            jnp.full((2 * N,), NE, jnp.int32)])
    _, dst0_s, eid0_s, offs0 = _sortset(None, dst0, eid0)

    e_p_row = jnp.zeros((1, 64), jnp.float32).at[0, 0].set(1.0)
    e_store0 = jnp.concatenate([e, e_p_row,
                                jnp.zeros((n_total + 7, 64), jnp.float32)])
    la = _sc_loop_attr(e_store0, dst0_s, eid0_s, offs0)
    loop_attr = la[:n_total, :64] / jnp.clip(la[:n_total, 64:65], 1.0, None)
    e_store = lax.dynamic_update_slice(e_store0, loop_attr, (NE + 1, 0))

    sl = jnp.arange(n_total, dtype=edge_index.dtype)
    aug_src = jnp.concatenate([src0, sl])
    aug_dst = jnp.concatenate([dst0, sl])
    eid = jnp.concatenate([eid0, NE + 1 + jnp.arange(n_total, dtype=jnp.int32)])
    src_s, dst_s, eid_s, offs = _sortset(aug_src, aug_dst, eid)

    H, C = NUM_HEADS, NUM_EMB
    for p in params["layers"]:
        o_pad = jnp.concatenate([o, jnp.zeros((NPAD - n_total, o.shape[1]), jnp.float32)])
        agg = _sc_gen_msg(o_pad, e_store, src_s, dst_s, eid_s, offs)[:n_total]
        genout = (agg + o) @ p["gen_W"] + p["gen_b"]

        xc = jnp.concatenate([o, genout], axis=1)
        q = xc @ p["Wq"] + p["bq"]
        k = xc @ p["Wk"] + p["bk"]
        vv = xc @ p["Wv"] + p["bv"]
        skip = xc @ p["Wsk"] + p["bsk"]
        We0 = p["We"][:, :64]
        We1 = p["We"][:, 64:]
        qWe = jnp.concatenate([q[:, :64] @ We0.T, q[:, 64:] @ We1.T], axis=1)
        elC = loop_attr @ p["We"]
        s0 = (q[:, :64] * (k[:, :64] + elC[:, :64])).sum(1) * 0.125
        s1 = (q[:, 64:] * (k[:, 64:] + elC[:, 64:])).sum(1) * 0.125
        dtab = jnp.concatenate([q, qWe, s0[:, None], s1[:, None],
                                jnp.zeros((n_total, 14), jnp.float32)], axis=1)
        dtab = jnp.concatenate([dtab, jnp.zeros((NPAD - n_total, 272), jnp.float32)])
        stab = jnp.concatenate([k, vv], axis=1)
        stab = jnp.concatenate([stab, jnp.zeros((NPAD - n_total, 256), jnp.float32)])
        accA = _sc_attn(dtab, stab, e_store, src_s, dst_s, eid_s, offs)
        vacc = accA[:n_total, 0:128]
        eacc0 = accA[:n_total, 128:192]
        eacc1 = accA[:n_total, 192:256]
        zsum = accA[:n_total, 256:258]
        out0 = (vacc[:, :64] + eacc0 @ We0) / zsum[:, 0:1]
        out1 = (vacc[:, 64:] + eacc1 @ We1) / zsum[:, 1:2]
        t = jnp.concatenate([out0, out1], axis=1) + skip

        o = _graph_layer_norm(o + (t @ p["lin_W"] + p["lin_b"]), aug_batch, G)
        o = _graph_layer_norm(o + _mlp_apply(o, p["ff"]), aug_batch, G)

    node_cnt = jax.ops.segment_sum(jnp.ones((N,), jnp.float32), batch, G)
    gmp = jax.ops.segment_sum(o[:N], batch, G) / jnp.clip(node_cnt, 1.0, None)[:, None]
    glob = jnp.concatenate([gmp, o[N:], c], axis=1)
    o_final = jnp.concatenate([o[:N], c[batch]], axis=1)
    return (o_final, glob)


# speedup vs baseline: 9.1093x; 1.2383x over previous
import math
from functools import partial

import jax
import jax.numpy as jnp
from jax import lax
from jax.experimental import pallas as pl
from jax.experimental.pallas import tpu as pltpu
from jax.experimental.pallas import tpu_sc as plsc

N_NODES = 10000
N_GRAPHS = 128
NUM_EMB = 64
NUM_HEADS = 2

NC, NS, LANES = 2, 16, 16
NW = NC * NS
NPAD = 10240
ROWS_W = NPAD // NW
EDGE_SLACK = 1280


def _sc_mesh():
    return plsc.VectorSubcoreMesh(core_axis_name="c", subcore_axis_name="s")


def _sc_params():
    return pltpu.CompilerParams(use_tc_tiling_on_sc=False,
                                needs_layout_passes=False)


def _zero_acc(acc, rows, width):
    z = jnp.zeros((LANES,), jnp.float32)

    @pl.loop(0, rows)
    def _(r):
        for f in range(width // LANES):
            acc[r, pl.ds(f * LANES, LANES)] = z


def _pipeline(ch, lo, hi, idx_streams, gath_streams, isems, gsems, compute):
    lo8 = (lo // 8) * 8
    nj = (hi - lo8 + ch - 1) // ch
    npair = (nj + 1) // 2

    def fire_idx(p, base):
        for arr, buf in idx_streams:
            pltpu.make_async_copy(arr.at[pl.ds(base, ch)], buf.at[p], isems[p]).start()

    def wait_idx(p):
        for arr, buf in idx_streams:
            pltpu.make_async_copy(arr.at[pl.ds(0, ch)], buf.at[p], isems[p]).wait()

    def fire_gath(p):
        for tab, ibuf, dbuf in gath_streams:
            pltpu.make_async_copy(tab.at[ibuf.at[p]], dbuf.at[p], gsems[p]).start()

    def wait_gath(p):
        for tab, ibuf, dbuf in gath_streams:
            pltpu.make_async_copy(tab.at[ibuf.at[p]], dbuf.at[p], gsems[p]).wait()

    fire_idx(0, lo8)
    wait_idx(0)
    fire_gath(0)
    fire_idx(1, lo8 + ch)

    @pl.loop(0, npair)
    def _(jj):
        b0 = lo8 + (2 * jj) * ch
        for p in (0, 1):
            base = b0 + p * ch
            wait_idx(1 - p)
            fire_gath(1 - p)
            wait_gath(p)
            compute(p, base, lo, hi)
            fire_idx(p, base + 2 * ch)

    wait_gath(0)
    wait_idx(1)


GEN_CH = 256


def _gen_msg_body(o_hbm, es_hbm, src_hbm, dst_hbm, eid_hbm, off_hbm, out_hbm,
                  sidx, didx, eidx, off_v, gbuf, ebuf, acc, si0, si1, sg0, sg1):
    c = lax.axis_index("c")
    s = lax.axis_index("s")
    w = c * NS + s
    row0 = w * ROWS_W

    _zero_acc(acc, ROWS_W, 64)
    pltpu.sync_copy(off_hbm, off_v)
    iota = lax.iota(jnp.int32, LANES)
    ovec = plsc.load_gather(off_v, [w + iota])

    def compute(p, base, lo, hi):
        @pl.loop(0, GEN_CH // LANES)
        def _(g):
            dvec = jnp.clip(didx[p, pl.ds(g * LANES, LANES)] - row0, 0, ROWS_W - 1)
            for i in range(LANES):
                r = g * LANES + i
                pos = base + r
                valid = jnp.logical_and(pos >= lo, pos < hi)
                m = jnp.broadcast_to(valid, (LANES,))
                rowv = jnp.broadcast_to(dvec[i], (LANES,))
                for f in range(4):
                    sl = pl.ds(f * LANES, LANES)
                    vv = jnp.maximum(gbuf[p, r, sl] + ebuf[p, r, sl], 0.0) + 1e-7
                    plsc.addupdate_scatter(acc, [rowv, iota + f * LANES], vv, mask=m)

    _pipeline(GEN_CH, ovec[0], ovec[1],
              [(src_hbm, sidx), (dst_hbm, didx), (eid_hbm, eidx)],
              [(o_hbm, sidx, gbuf), (es_hbm, eidx, ebuf)],
              (si0, si1), (sg0, sg1), compute)

    pltpu.sync_copy(acc, out_hbm.at[pl.ds(row0, ROWS_W)])


@jax.jit
def _sc_gen_msg(o_pad, e_store, src_s, dst_s, eid_s, offs):
    k = pl.kernel(
        _gen_msg_body,
        out_type=jax.ShapeDtypeStruct((NPAD, 64), jnp.float32),
        mesh=_sc_mesh(),
        compiler_params=_sc_params(),
        scratch_types=[
            pltpu.VMEM((2, GEN_CH), jnp.int32),
            pltpu.VMEM((2, GEN_CH), jnp.int32),
            pltpu.VMEM((2, GEN_CH), jnp.int32),
            pltpu.VMEM((48,), jnp.int32),
            pltpu.VMEM((2, GEN_CH, 64), jnp.float32),
            pltpu.VMEM((2, GEN_CH, 64), jnp.float32),
            pltpu.VMEM((ROWS_W, 64), jnp.float32),
            pltpu.SemaphoreType.DMA,
            pltpu.SemaphoreType.DMA,
            pltpu.SemaphoreType.DMA,
            pltpu.SemaphoreType.DMA,
        ],
    )
    return k(o_pad, e_store, src_s, dst_s, eid_s, offs)


BC_CH = 64
HALF_W = ROWS_W // 2


def _attn_body(dtab, stab, es_hbm, src_hbm, dst_hbm, eid_hbm, off_hbm, out_hbm,
               sidx, didx, eidx, off_v, dgath, sgath, ebuf, acc,
               si0, si1, sg0, sg1):
    c = lax.axis_index("c")
    s = lax.axis_index("s")
    w = c * NS + s

    pltpu.sync_copy(off_hbm, off_v)
    iota = lax.iota(jnp.int32, LANES)

    for half in range(2):
        row0 = w * ROWS_W + half * HALF_W
        _zero_acc(acc, HALF_W, 272)
        ovec = plsc.load_gather(off_v, [2 * w + half + iota])

        def compute(p, base, lo, hi):
            @pl.loop(0, BC_CH // LANES)
            def _(g):
                dvec = jnp.clip(didx[p, pl.ds(g * LANES, LANES)] - row0, 0, HALF_W - 1)
                for i in range(LANES):
                    r = g * LANES + i
                    pos = base + r
                    valid = jnp.logical_and(pos >= lo, pos < hi)
                    m = jnp.broadcast_to(valid, (LANES,))
                    rowv = jnp.broadcast_to(dvec[i], (LANES,))
                    qk = [dgath[p, r, pl.ds(f * LANES, LANES)] for f in range(8)]
                    qe = [dgath[p, r, pl.ds(128 + f * LANES, LANES)] for f in range(8)]
                    kv = [sgath[p, r, pl.ds(f * LANES, LANES)] for f in range(8)]
                    vv = [sgath[p, r, pl.ds(128 + f * LANES, LANES)] for f in range(8)]
                    ev = [ebuf[p, r, pl.ds(f * LANES, LANES)] for f in range(4)]
                    svec = dgath[p, r, pl.ds(256, LANES)]
                    m0 = qk[0] * kv[0]
                    m1 = qk[4] * kv[4]
                    for f in range(1, 4):
                        m0 = m0 + qk[f] * kv[f]
                        m1 = m1 + qk[4 + f] * kv[4 + f]
                    for f in range(4):
                        m0 = m0 + qe[f] * ev[f]
                        m1 = m1 + qe[4 + f] * ev[f]
                    a0 = jnp.sum(m0) * 0.125 - svec[0]
                    a1 = jnp.sum(m1) * 0.125 - svec[1]
                    zb0 = jnp.exp(jnp.broadcast_to(a0, (LANES,)))
                    zb1 = jnp.exp(jnp.broadcast_to(a1, (LANES,)))
                    for f in range(4):
                        sl = iota + f * LANES
                        plsc.addupdate_scatter(acc, [rowv, sl], vv[f] * zb0, mask=m)
                        plsc.addupdate_scatter(acc, [rowv, sl + 64], vv[4 + f] * zb1, mask=m)
                        plsc.addupdate_scatter(acc, [rowv, sl + 128], ev[f] * zb0, mask=m)
                        plsc.addupdate_scatter(acc, [rowv, sl + 192], ev[f] * zb1, mask=m)
                    zrow = jnp.where(iota == 0, zb0, jnp.where(iota == 1, zb1, 0.0))
                    plsc.addupdate_scatter(acc, [rowv, iota + 256], zrow, mask=m)

        _pipeline(BC_CH, ovec[0], ovec[1],
                  [(src_hbm, sidx), (dst_hbm, didx), (eid_hbm, eidx)],
                  [(dtab, didx, dgath), (stab, sidx, sgath), (es_hbm, eidx, ebuf)],
                  (si0, si1), (sg0, sg1), compute)

        pltpu.sync_copy(acc, out_hbm.at[pl.ds(row0, HALF_W)])


@jax.jit
def _sc_attn(dtab, stab, e_store, src_s, dst_s, eid_s, offs64):
    k = pl.kernel(
        _attn_body,
        out_type=jax.ShapeDtypeStruct((NPAD, 272), jnp.float32),
        mesh=_sc_mesh(),
        compiler_params=_sc_params(),
        scratch_types=[
            pltpu.VMEM((2, BC_CH), jnp.int32),
            pltpu.VMEM((2, BC_CH), jnp.int32),
            pltpu.VMEM((2, BC_CH), jnp.int32),
            pltpu.VMEM((80,), jnp.int32),
            pltpu.VMEM((2, BC_CH, 272), jnp.float32),
            pltpu.VMEM((2, BC_CH, 256), jnp.float32),
            pltpu.VMEM((2, BC_CH, 64), jnp.float32),
            pltpu.VMEM((HALF_W, 272), jnp.float32),
            pltpu.SemaphoreType.DMA,
            pltpu.SemaphoreType.DMA,
            pltpu.SemaphoreType.DMA,
            pltpu.SemaphoreType.DMA,
        ],
    )
    return k(dtab, stab, e_store, src_s, dst_s, eid_s, offs64)


LA_CH = 256


def _lattr_body(es_hbm, dst_hbm, eid_hbm, off_hbm, out_hbm,
                didx, eidx, off_v, ebuf, acc, si0, si1, sg0, sg1):
    c = lax.axis_index("c")
    s = lax.axis_index("s")
    w = c * NS + s
    row0 = w * ROWS_W

    _zero_acc(acc, ROWS_W, 80)
    pltpu.sync_copy(off_hbm, off_v)
    iota = lax.iota(jnp.int32, LANES)
    ovec = plsc.load_gather(off_v, [w + iota])
    onerow = jnp.where(iota == 0, 1.0, 0.0)

    def compute(p, base, lo, hi):
        @pl.loop(0, LA_CH // LANES)
        def _(g):
            dvec = jnp.clip(didx[p, pl.ds(g * LANES, LANES)] - row0, 0, ROWS_W - 1)
            for i in range(LANES):
                r = g * LANES + i
                pos = base + r
                valid = jnp.logical_and(pos >= lo, pos < hi)
                m = jnp.broadcast_to(valid, (LANES,))
                rowv = jnp.broadcast_to(dvec[i], (LANES,))
                for f in range(4):
                    plsc.addupdate_scatter(acc, [rowv, iota + f * LANES],
                                           ebuf[p, r, pl.ds(f * LANES, LANES)], mask=m)
                plsc.addupdate_scatter(acc, [rowv, iota + 64], onerow, mask=m)

    _pipeline(LA_CH, ovec[0], ovec[1],
              [(dst_hbm, didx), (eid_hbm, eidx)],
              [(es_hbm, eidx, ebuf)],
              (si0, si1), (sg0, sg1), compute)

    pltpu.sync_copy(acc, out_hbm.at[pl.ds(row0, ROWS_W)])


@jax.jit
def _sc_loop_attr(e_store, dst_s, eid_s, offs):
    k = pl.kernel(
        _lattr_body,
        out_type=jax.ShapeDtypeStruct((NPAD, 80), jnp.float32),
        mesh=_sc_mesh(),
        compiler_params=_sc_params(),
        scratch_types=[
            pltpu.VMEM((2, LA_CH), jnp.int32),
            pltpu.VMEM((2, LA_CH), jnp.int32),
            pltpu.VMEM((48,), jnp.int32),
            pltpu.VMEM((2, LA_CH, 64), jnp.float32),
            pltpu.VMEM((ROWS_W, 80), jnp.float32),
            pltpu.SemaphoreType.DMA,
            pltpu.SemaphoreType.DMA,
            pltpu.SemaphoreType.DMA,
            pltpu.SemaphoreType.DMA,
        ],
    )
    return k(e_store, dst_s, eid_s, offs)


def _leaky(x):
    return jnp.where(x >= 0, x, 0.01 * x)


def _mlp3_body(a_ref, w1, b1, w2, b2, w3, b3, o_ref):
    a = a_ref[...]
    h = _leaky(jnp.dot(a, w1[...], preferred_element_type=jnp.float32) + b1[...])
    h = _leaky(jnp.dot(h, w2[...], preferred_element_type=jnp.float32) + b2[...])
    h = jnp.dot(h, w3[...], preferred_element_type=jnp.float32) + b3[...]
    o_ref[...] = h


def _mlp3(a, p, bm=2048):
    M, F = a.shape
    D = p["Ws"][2].shape[1]
    grid = (pl.cdiv(M, bm),)
    full = lambda shape: pl.BlockSpec(shape, lambda i: (0,) * len(shape))
    return pl.pallas_call(
        _mlp3_body,
        grid=grid,
        in_specs=[
            pl.BlockSpec((bm, F), lambda i: (i, 0)),
            full(p["Ws"][0].shape), full((1, p["bs"][0].shape[0])),
            full(p["Ws"][1].shape), full((1, p["bs"][1].shape[0])),
            full(p["Ws"][2].shape), full((1, p["bs"][2].shape[0])),
        ],
        out_specs=pl.BlockSpec((bm, D), lambda i: (i, 0)),
        out_shape=jax.ShapeDtypeStruct((M, D), jnp.float32),
    )(a, p["Ws"][0], p["bs"][0][None], p["Ws"][1], p["bs"][1][None],
      p["Ws"][2], p["bs"][2][None])


def _graph_layer_norm(x, batch, num_graphs, eps=1e-5):
    cnt = jax.ops.segment_sum(jnp.ones((x.shape[0],), jnp.float32), batch, num_graphs)
    norm = jnp.clip(cnt, 1.0, None) * x.shape[1]
    mean = jax.ops.segment_sum(x, batch, num_graphs).sum(axis=-1) / norm
    x = x - mean[batch][:, None]
    var = jax.ops.segment_sum(x * x, batch, num_graphs).sum(axis=-1) / norm
    return x / jnp.sqrt(var + eps)[batch][:, None]


def _mlp_apply(x, p):
    n = len(p["Ws"])
    for i in range(n):
        x = x @ p["Ws"][i] + p["bs"][i]
        if i < n - 1:
            x = _leaky(x)
    return x


def kernel(x, edge_index, edge_attr, batch, cond, params):
    N, G = x.shape[0], cond.shape[0]
    o = _mlp3(x, params["x2h"])
    e = _mlp3(edge_attr, params["e2h"])
    c = _mlp3(cond, params["c2h"], bm=128)

    u = jnp.arange(N, dtype=edge_index.dtype)
    v = batch.astype(edge_index.dtype) + N
    n_total = N + G
    NE = edge_index.shape[1]
    aug_batch = jnp.concatenate([batch, jnp.arange(G, dtype=batch.dtype)])
    o = jnp.concatenate([o, c], axis=0)

    def _sortset(srcs, dsts, eids, gran):
        E = dsts.shape[0]
        E_pad = 128 * pl.cdiv(E, 128) + EDGE_SLACK
        pad_e = E_pad - E
        perm = jnp.argsort(dsts)

        def _padi(a):
            return jnp.concatenate([a.astype(jnp.int32),
                                    jnp.zeros((pad_e,), jnp.int32)])

        dst_sorted = dsts[perm]
        nb = NPAD // gran
        offs = jnp.searchsorted(dst_sorted,
                                jnp.arange(nb + 1, dtype=jnp.int32) * gran)
        npadofs = 16 * pl.cdiv(nb + 16, 16)
        offs = jnp.concatenate([
            offs.astype(jnp.int32),
            jnp.full((npadofs - nb - 1,), E, jnp.int32),
        ])
        src_p = _padi(srcs[perm]) if srcs is not None else None
        return src_p, _padi(dst_sorted), _padi(eids[perm]), offs

    src0 = jnp.concatenate([edge_index[0], u, v])
    dst0 = jnp.concatenate([edge_index[1], v, u])
    eid0 = jnp.concatenate([jnp.arange(NE, dtype=jnp.int32),
                            jnp.full((2 * N,), NE, jnp.int32)])
    _, dst0_s, eid0_s, offs0 = _sortset(None, dst0, eid0, ROWS_W)

    e_p_row = jnp.zeros((1, 64), jnp.float32).at[0, 0].set(1.0)
    e_store0 = jnp.concatenate([e, e_p_row,
                                jnp.zeros((n_total + 7, 64), jnp.float32)])
    la = _sc_loop_attr(e_store0, dst0_s, eid0_s, offs0)
    loop_attr = la[:n_total, :64] / jnp.clip(la[:n_total, 64:65], 1.0, None)
    e_store = lax.dynamic_update_slice(e_store0, loop_attr, (NE + 1, 0))

    sl = jnp.arange(n_total, dtype=edge_index.dtype)
    aug_src = jnp.concatenate([src0, sl])
    aug_dst = jnp.concatenate([dst0, sl])
    eid = jnp.concatenate([eid0, NE + 1 + jnp.arange(n_total, dtype=jnp.int32)])
    src_s, dst_s, eid_s, offs = _sortset(aug_src, aug_dst, eid, ROWS_W)
    _, _, _, offs64 = _sortset(None, aug_dst, eid, HALF_W)

    H, C = NUM_HEADS, NUM_EMB
    for p in params["layers"]:
        o_pad = jnp.concatenate([o, jnp.zeros((NPAD - n_total, o.shape[1]), jnp.float32)])
        agg = _sc_gen_msg(o_pad, e_store, src_s, dst_s, eid_s, offs)[:n_total]
        genout = (agg + o) @ p["gen_W"] + p["gen_b"]

        xc = jnp.concatenate([o, genout], axis=1)
        q = xc @ p["Wq"] + p["bq"]
        k = xc @ p["Wk"] + p["bk"]
        vv = xc @ p["Wv"] + p["bv"]
        skip = xc @ p["Wsk"] + p["bsk"]
        We0 = p["We"][:, :64]
        We1 = p["We"][:, 64:]
        qWe = jnp.concatenate([q[:, :64] @ We0.T, q[:, 64:] @ We1.T], axis=1)
        elC = loop_attr @ p["We"]
        s0 = (q[:, :64] * (k[:, :64] + elC[:, :64])).sum(1) * 0.125
        s1 = (q[:, 64:] * (k[:, 64:] + elC[:, 64:])).sum(1) * 0.125
        dtab = jnp.concatenate([q, qWe, s0[:, None], s1[:, None],
                                jnp.zeros((n_total, 14), jnp.float32)], axis=1)
        dtab = jnp.concatenate([dtab, jnp.zeros((NPAD - n_total, 272), jnp.float32)])
        stab = jnp.concatenate([k, vv], axis=1)
        stab = jnp.concatenate([stab, jnp.zeros((NPAD - n_total, 256), jnp.float32)])
        accA = _sc_attn(dtab, stab, e_store, src_s, dst_s, eid_s, offs64)
        vacc = accA[:n_total, 0:128]
        eacc0 = accA[:n_total, 128:192]
        eacc1 = accA[:n_total, 192:256]
        zsum = accA[:n_total, 256:258]
        out0 = (vacc[:, :64] + eacc0 @ We0) / zsum[:, 0:1]
        out1 = (vacc[:, 64:] + eacc1 @ We1) / zsum[:, 1:2]
        t = jnp.concatenate([out0, out1], axis=1) + skip

        o = _graph_layer_norm(o + (t @ p["lin_W"] + p["lin_b"]), aug_batch, G)
        o = _graph_layer_norm(o + _mlp_apply(o, p["ff"]), aug_batch, G)

    node_cnt = jax.ops.segment_sum(jnp.ones((N,), jnp.float32), batch, G)
    gmp = jax.ops.segment_sum(o[:N], batch, G) / jnp.clip(node_cnt, 1.0, None)[:, None]
    glob = jnp.concatenate([gmp, o[N:], c], axis=1)
    o_final = jnp.concatenate([o[:N], c[batch]], axis=1)
    return (o_final, glob)


# speedup vs baseline: 11.7737x; 1.2925x over previous
import math
from functools import partial

import jax
import jax.numpy as jnp
from jax import lax
from jax.experimental import pallas as pl
from jax.experimental.pallas import tpu as pltpu
from jax.experimental.pallas import tpu_sc as plsc

N_NODES = 10000
N_GRAPHS = 128
NUM_EMB = 64
NUM_HEADS = 2

NC, NS, LANES = 2, 16, 16
NW = NC * NS
NPAD = 10240
ROWS_W = NPAD // NW
EDGE_SLACK = 1280


def _sc_mesh():
    return plsc.VectorSubcoreMesh(core_axis_name="c", subcore_axis_name="s")


def _sc_params():
    return pltpu.CompilerParams(use_tc_tiling_on_sc=False,
                                needs_layout_passes=False)


def _zero_acc(acc, rows, width):
    z = jnp.zeros((LANES,), jnp.float32)

    @pl.loop(0, rows)
    def _(r):
        for f in range(width // LANES):
            acc[r, pl.ds(f * LANES, LANES)] = z


def _pipeline(ch, lo, hi, idx_streams, gath_streams, isems, gsems, compute):
    lo8 = (lo // 8) * 8
    nj = (hi - lo8 + ch - 1) // ch
    npair = (nj + 1) // 2

    def fire_idx(p, base):
        for arr, buf in idx_streams:
            pltpu.make_async_copy(arr.at[pl.ds(base, ch)], buf.at[p], isems[p]).start()

    def wait_idx(p):
        for arr, buf in idx_streams:
            pltpu.make_async_copy(arr.at[pl.ds(0, ch)], buf.at[p], isems[p]).wait()

    def fire_gath(p):
        for tab, ibuf, dbuf in gath_streams:
            pltpu.make_async_copy(tab.at[ibuf.at[p]], dbuf.at[p], gsems[p]).start()

    def wait_gath(p):
        for tab, ibuf, dbuf in gath_streams:
            pltpu.make_async_copy(tab.at[ibuf.at[p]], dbuf.at[p], gsems[p]).wait()

    fire_idx(0, lo8)
    wait_idx(0)
    fire_gath(0)
    fire_idx(1, lo8 + ch)

    @pl.loop(0, npair)
    def _(jj):
        b0 = lo8 + (2 * jj) * ch
        for p in (0, 1):
            base = b0 + p * ch
            wait_idx(1 - p)
            fire_gath(1 - p)
            wait_gath(p)
            compute(p, base, lo, hi)
            fire_idx(p, base + 2 * ch)

    wait_gath(0)
    wait_idx(1)


GEN_CH = 256


def _gen_msg_body(o_hbm, es_hbm, src_hbm, dst_hbm, eid_hbm, off_hbm, out_hbm,
                  sidx, didx, eidx, off_v, gbuf, ebuf, acc, si0, si1, sg0, sg1):
    c = lax.axis_index("c")
    s = lax.axis_index("s")
    w = c * NS + s
    row0 = w * ROWS_W

    _zero_acc(acc, ROWS_W, 64)
    pltpu.sync_copy(off_hbm, off_v)
    iota = lax.iota(jnp.int32, LANES)
    ovec = plsc.load_gather(off_v, [w + iota])

    def compute(p, base, lo, hi):
        @pl.loop(0, GEN_CH // LANES)
        def _(g):
            dvec = jnp.clip(didx[p, pl.ds(g * LANES, LANES)] - row0, 0, ROWS_W - 1)
            for i in range(LANES):
                r = g * LANES + i
                pos = base + r
                valid = jnp.logical_and(pos >= lo, pos < hi)
                m = jnp.broadcast_to(valid, (LANES,))
                rowv = jnp.broadcast_to(dvec[i], (LANES,))
                for f in range(4):
                    sl = pl.ds(f * LANES, LANES)
                    vv = jnp.maximum(gbuf[p, r, sl] + ebuf[p, r, sl], 0.0) + 1e-7
                    plsc.addupdate_scatter(acc, [rowv, iota + f * LANES], vv, mask=m)

    _pipeline(GEN_CH, ovec[0], ovec[1],
              [(src_hbm, sidx), (dst_hbm, didx), (eid_hbm, eidx)],
              [(o_hbm, sidx, gbuf), (es_hbm, eidx, ebuf)],
              (si0, si1), (sg0, sg1), compute)

    pltpu.sync_copy(acc, out_hbm.at[pl.ds(row0, ROWS_W)])


@jax.jit
def _sc_gen_msg(o_pad, e_store, src_s, dst_s, eid_s, offs):
    k = pl.kernel(
        _gen_msg_body,
        out_type=jax.ShapeDtypeStruct((NPAD, 64), jnp.float32),
        mesh=_sc_mesh(),
        compiler_params=_sc_params(),
        scratch_types=[
            pltpu.VMEM((2, GEN_CH), jnp.int32),
            pltpu.VMEM((2, GEN_CH), jnp.int32),
            pltpu.VMEM((2, GEN_CH), jnp.int32),
            pltpu.VMEM((48,), jnp.int32),
            pltpu.VMEM((2, GEN_CH, 64), jnp.float32),
            pltpu.VMEM((2, GEN_CH, 64), jnp.float32),
            pltpu.VMEM((ROWS_W, 64), jnp.float32),
            pltpu.SemaphoreType.DMA,
            pltpu.SemaphoreType.DMA,
            pltpu.SemaphoreType.DMA,
            pltpu.SemaphoreType.DMA,
        ],
    )
    return k(o_pad, e_store, src_s, dst_s, eid_s, offs)


BC_CH = 64
HALF_W = ROWS_W // 2


def _attn_body(dtab, stab, es_hbm, src_hbm, dst_hbm, eid_hbm, off_hbm, out_hbm,
               sidx, didx, eidx, off_v, dgath, sgath, ebuf, acc,
               si0, si1, sg0, sg1):
    c = lax.axis_index("c")
    s = lax.axis_index("s")
    w = c * NS + s

    pltpu.sync_copy(off_hbm, off_v)
    iota = lax.iota(jnp.int32, LANES)

    for half in range(2):
        row0 = w * ROWS_W + half * HALF_W
        _zero_acc(acc, HALF_W, 272)
        ovec = plsc.load_gather(off_v, [2 * w + half + iota])

        def compute(p, base, lo, hi):
            @pl.loop(0, BC_CH // LANES)
            def _(g):
                dvec = jnp.clip(didx[p, pl.ds(g * LANES, LANES)] - row0, 0, HALF_W - 1)
                for i in range(LANES):
                    r = g * LANES + i
                    pos = base + r
                    valid = jnp.logical_and(pos >= lo, pos < hi)
                    m = jnp.broadcast_to(valid, (LANES,))
                    rowv = jnp.broadcast_to(dvec[i], (LANES,))
                    qk = [dgath[p, r, pl.ds(f * LANES, LANES)] for f in range(8)]
                    qe = [dgath[p, r, pl.ds(128 + f * LANES, LANES)] for f in range(8)]
                    kv = [sgath[p, r, pl.ds(f * LANES, LANES)] for f in range(8)]
                    vv = [sgath[p, r, pl.ds(128 + f * LANES, LANES)] for f in range(8)]
                    ev = [ebuf[p, r, pl.ds(f * LANES, LANES)] for f in range(4)]
                    svec = dgath[p, r, pl.ds(256, LANES)]
                    m0 = qk[0] * kv[0]
                    m1 = qk[4] * kv[4]
                    for f in range(1, 4):
                        m0 = m0 + qk[f] * kv[f]
                        m1 = m1 + qk[4 + f] * kv[4 + f]
                    for f in range(4):
                        m0 = m0 + qe[f] * ev[f]
                        m1 = m1 + qe[4 + f] * ev[f]
                    a0 = jnp.sum(m0) * 0.125 - svec[0]
                    a1 = jnp.sum(m1) * 0.125 - svec[1]
                    zb0 = jnp.exp(jnp.broadcast_to(a0, (LANES,)))
                    zb1 = jnp.exp(jnp.broadcast_to(a1, (LANES,)))
                    for f in range(4):
                        sl = iota + f * LANES
                        plsc.addupdate_scatter(acc, [rowv, sl], vv[f] * zb0, mask=m)
                        plsc.addupdate_scatter(acc, [rowv, sl + 64], vv[4 + f] * zb1, mask=m)
                        plsc.addupdate_scatter(acc, [rowv, sl + 128], ev[f] * zb0, mask=m)
                        plsc.addupdate_scatter(acc, [rowv, sl + 192], ev[f] * zb1, mask=m)
                    zrow = jnp.where(iota == 0, zb0, jnp.where(iota == 1, zb1, 0.0))
                    plsc.addupdate_scatter(acc, [rowv, iota + 256], zrow, mask=m)

        _pipeline(BC_CH, ovec[0], ovec[1],
                  [(src_hbm, sidx), (dst_hbm, didx), (eid_hbm, eidx)],
                  [(dtab, didx, dgath), (stab, sidx, sgath), (es_hbm, eidx, ebuf)],
                  (si0, si1), (sg0, sg1), compute)

        pltpu.sync_copy(acc, out_hbm.at[pl.ds(row0, HALF_W)])


@jax.jit
def _sc_attn(dtab, stab, e_store, src_s, dst_s, eid_s, offs64):
    k = pl.kernel(
        _attn_body,
        out_type=jax.ShapeDtypeStruct((NPAD, 272), jnp.float32),
        mesh=_sc_mesh(),
        compiler_params=_sc_params(),
        scratch_types=[
            pltpu.VMEM((2, BC_CH), jnp.int32),
            pltpu.VMEM((2, BC_CH), jnp.int32),
            pltpu.VMEM((2, BC_CH), jnp.int32),
            pltpu.VMEM((80,), jnp.int32),
            pltpu.VMEM((2, BC_CH, 272), jnp.float32),
            pltpu.VMEM((2, BC_CH, 256), jnp.float32),
            pltpu.VMEM((2, BC_CH, 64), jnp.float32),
            pltpu.VMEM((HALF_W, 272), jnp.float32),
            pltpu.SemaphoreType.DMA,
            pltpu.SemaphoreType.DMA,
            pltpu.SemaphoreType.DMA,
            pltpu.SemaphoreType.DMA,
        ],
    )
    return k(dtab, stab, e_store, src_s, dst_s, eid_s, offs64)


LA_CH = 256


def _lattr_body(es_hbm, dst_hbm, eid_hbm, off_hbm, out_hbm,
                didx, eidx, off_v, ebuf, acc, si0, si1, sg0, sg1):
    c = lax.axis_index("c")
    s = lax.axis_index("s")
    w = c * NS + s
    row0 = w * ROWS_W

    _zero_acc(acc, ROWS_W, 80)
    pltpu.sync_copy(off_hbm, off_v)
    iota = lax.iota(jnp.int32, LANES)
    ovec = plsc.load_gather(off_v, [w + iota])
    onerow = jnp.where(iota == 0, 1.0, 0.0)

    def compute(p, base, lo, hi):
        @pl.loop(0, LA_CH // LANES)
        def _(g):
            dvec = jnp.clip(didx[p, pl.ds(g * LANES, LANES)] - row0, 0, ROWS_W - 1)
            for i in range(LANES):
                r = g * LANES + i
                pos = base + r
                valid = jnp.logical_and(pos >= lo, pos < hi)
                m = jnp.broadcast_to(valid, (LANES,))
                rowv = jnp.broadcast_to(dvec[i], (LANES,))
                for f in range(4):
                    plsc.addupdate_scatter(acc, [rowv, iota + f * LANES],
                                           ebuf[p, r, pl.ds(f * LANES, LANES)], mask=m)
                plsc.addupdate_scatter(acc, [rowv, iota + 64], onerow, mask=m)

    _pipeline(LA_CH, ovec[0], ovec[1],
              [(dst_hbm, didx), (eid_hbm, eidx)],
              [(es_hbm, eidx, ebuf)],
              (si0, si1), (sg0, sg1), compute)

    pltpu.sync_copy(acc, out_hbm.at[pl.ds(row0, ROWS_W)])


@jax.jit
def _sc_loop_attr(e_store, dst_s, eid_s, offs):
    k = pl.kernel(
        _lattr_body,
        out_type=jax.ShapeDtypeStruct((NPAD, 80), jnp.float32),
        mesh=_sc_mesh(),
        compiler_params=_sc_params(),
        scratch_types=[
            pltpu.VMEM((2, LA_CH), jnp.int32),
            pltpu.VMEM((2, LA_CH), jnp.int32),
            pltpu.VMEM((48,), jnp.int32),
            pltpu.VMEM((2, LA_CH, 64), jnp.float32),
            pltpu.VMEM((ROWS_W, 80), jnp.float32),
            pltpu.SemaphoreType.DMA,
            pltpu.SemaphoreType.DMA,
            pltpu.SemaphoreType.DMA,
            pltpu.SemaphoreType.DMA,
        ],
    )
    return k(e_store, dst_s, eid_s, offs)


def _leaky(x):
    return jnp.where(x >= 0, x, 0.01 * x)


def _mlp3_body(a_ref, w1, b1, w2, b2, w3, b3, o_ref):
    a = a_ref[...]
    h = _leaky(jnp.dot(a, w1[...], preferred_element_type=jnp.float32) + b1[...])
    h = _leaky(jnp.dot(h, w2[...], preferred_element_type=jnp.float32) + b2[...])
    h = jnp.dot(h, w3[...], preferred_element_type=jnp.float32) + b3[...]
    o_ref[...] = h


def _mlp3(a, p, bm=2048):
    M, F = a.shape
    D = p["Ws"][2].shape[1]
    grid = (pl.cdiv(M, bm),)
    full = lambda shape: pl.BlockSpec(shape, lambda i: (0,) * len(shape))
    return pl.pallas_call(
        _mlp3_body,
        grid=grid,
        in_specs=[
            pl.BlockSpec((bm, F), lambda i: (i, 0)),
            full(p["Ws"][0].shape), full((1, p["bs"][0].shape[0])),
            full(p["Ws"][1].shape), full((1, p["bs"][1].shape[0])),
            full(p["Ws"][2].shape), full((1, p["bs"][2].shape[0])),
        ],
        out_specs=pl.BlockSpec((bm, D), lambda i: (i, 0)),
        out_shape=jax.ShapeDtypeStruct((M, D), jnp.float32),
    )(a, p["Ws"][0], p["bs"][0][None], p["Ws"][1], p["bs"][1][None],
      p["Ws"][2], p["bs"][2][None])


BM = 1024


def _full(shape):
    return pl.BlockSpec(shape, lambda i: (0,) * len(shape))


def _rows(width):
    return pl.BlockSpec((BM, width), lambda i: (i, 0))


def _t1_body(o_ref, agg_ref, la_ref, genW, genb, wqkvs, bqkvs, weT2, we,
             dtab_ref, stab_ref, skip_ref):
    o = o_ref[...]
    genout = jnp.dot(agg_ref[...] + o, genW[...],
                     preferred_element_type=jnp.float32) + genb[...]
    xc = jnp.concatenate([o, genout], axis=1)
    qkvs = jnp.dot(xc, wqkvs[...], preferred_element_type=jnp.float32) + bqkvs[...]
    q = qkvs[:, 0:128]
    k = qkvs[:, 128:256]
    vv = qkvs[:, 256:384]
    skip_ref[...] = qkvs[:, 384:512]
    qWe = jnp.dot(q, weT2[...], preferred_element_type=jnp.float32)
    elC = jnp.dot(la_ref[...], we[...], preferred_element_type=jnp.float32)
    kc = k + elC
    s0 = (q[:, :64] * kc[:, :64]).sum(axis=1, keepdims=True) * 0.125
    s1 = (q[:, 64:] * kc[:, 64:]).sum(axis=1, keepdims=True) * 0.125
    dtab_ref[...] = jnp.concatenate(
        [q, qWe, s0, s1, jnp.zeros((o.shape[0], 14), jnp.float32)], axis=1)
    stab_ref[...] = jnp.concatenate([k, vv], axis=1)


def _t1(o, agg, la_pad, p):
    wqkvs = jnp.concatenate([p["Wq"], p["Wk"], p["Wv"], p["Wsk"]], axis=1)
    bqkvs = jnp.concatenate([p["bq"], p["bk"], p["bv"], p["bsk"]])[None]
    z64 = jnp.zeros((64, 64), jnp.float32)
    weT2 = jnp.concatenate([
        jnp.concatenate([p["We"][:, :64].T, z64], axis=1),
        jnp.concatenate([z64, p["We"][:, 64:].T], axis=1)], axis=0)
    return pl.pallas_call(
        _t1_body,
        grid=(NPAD // BM,),
        in_specs=[_rows(64), _rows(64), _rows(64),
                  _full((64, 64)), _full((1, 64)), _full((128, 512)),
                  _full((1, 512)), _full((128, 128)), _full((64, 128))],
        out_specs=[_rows(272), _rows(256), _rows(128)],
        out_shape=[jax.ShapeDtypeStruct((NPAD, 272), jnp.float32),
                   jax.ShapeDtypeStruct((NPAD, 256), jnp.float32),
                   jax.ShapeDtypeStruct((NPAD, 128), jnp.float32)],
    )(o, agg, la_pad, p["gen_W"], p["gen_b"][None], wqkvs, bqkvs, weT2, p["We"])


def _t2_body(acc_ref, skip_ref, o_ref, we0, we1, linW, linb, pt_ref,
             y_ref, stats_ref):
    i = pl.program_id(0)
    a = acc_ref[...]
    out0 = (a[:, 0:64] + jnp.dot(a[:, 128:192], we0[...],
                                 preferred_element_type=jnp.float32)) / a[:, 256:257]
    out1 = (a[:, 64:128] + jnp.dot(a[:, 192:256], we1[...],
                                   preferred_element_type=jnp.float32)) / a[:, 257:258]
    t = jnp.concatenate([out0, out1], axis=1) + skip_ref[...]
    y = o_ref[...] + jnp.dot(t, linW[...], preferred_element_type=jnp.float32) + linb[...]
    y_ref[...] = y
    yy = jnp.concatenate([y, y * y, jnp.ones((y.shape[0], 16), jnp.float32)], axis=1)
    st = jnp.dot(pt_ref[...], yy, preferred_element_type=jnp.float32)

    @pl.when(i == 0)
    def _():
        stats_ref[...] = jnp.zeros_like(stats_ref)

    stats_ref[...] += st


def _t2(accA, skip, o, pt, p):
    return pl.pallas_call(
        _t2_body,
        grid=(NPAD // BM,),
        in_specs=[_rows(272), _rows(128), _rows(64),
                  _full((64, 64)), _full((64, 64)), _full((128, 64)),
                  _full((1, 64)), pl.BlockSpec((128, BM), lambda i: (0, i))],
        out_specs=[_rows(64), _full((128, 144))],
        out_shape=[jax.ShapeDtypeStruct((NPAD, 64), jnp.float32),
                   jax.ShapeDtypeStruct((128, 144), jnp.float32)],
    )(accA, skip, o, p["We"][:, :64], p["We"][:, 64:], p["lin_W"],
      p["lin_b"][None], pt)


def _stats_to_mi(stats, eps=1e-5):
    s1 = stats[:, 0:64].sum(axis=1)
    s2 = stats[:, 64:128].sum(axis=1)
    cnt = stats[:, 128]
    norm = jnp.maximum(cnt, 1.0) * 64.0
    mean = s1 / norm
    var = s2 / norm - mean * mean
    inv = 1.0 / jnp.sqrt(var + eps)
    z = jnp.zeros((128, 126), jnp.float32)
    return jnp.concatenate([mean[:, None], inv[:, None], z], axis=1)


def _t3_body(y_ref, stats_ref, pb_ref, pt_ref, w1, b1, w2, b2,
             y2_ref, stats2_ref):
    i = pl.program_id(0)
    mi = jnp.dot(pb_ref[...], _stats_to_mi(stats_ref[...]),
                 preferred_element_type=jnp.float32)
    o1 = (y_ref[...] - mi[:, 0:1]) * mi[:, 1:2]
    h = _leaky(jnp.dot(o1, w1[...], preferred_element_type=jnp.float32) + b1[...])
    y2 = o1 + jnp.dot(h, w2[...], preferred_element_type=jnp.float32) + b2[...]
    y2_ref[...] = y2
    yy = jnp.concatenate([y2, y2 * y2, jnp.ones((y2.shape[0], 16), jnp.float32)], axis=1)
    st = jnp.dot(pt_ref[...], yy, preferred_element_type=jnp.float32)

    @pl.when(i == 0)
    def _():
        stats2_ref[...] = jnp.zeros_like(stats2_ref)

    stats2_ref[...] += st


def _t3(y, stats, pmat, pt, p):
    return pl.pallas_call(
        _t3_body,
        grid=(NPAD // BM,),
        in_specs=[_rows(64), _full((128, 144)), _rows(128),
                  pl.BlockSpec((128, BM), lambda i: (0, i)),
                  _full((64, 256)), _full((1, 256)), _full((256, 64)),
                  _full((1, 64))],
        out_specs=[_rows(64), _full((128, 144))],
        out_shape=[jax.ShapeDtypeStruct((NPAD, 64), jnp.float32),
                   jax.ShapeDtypeStruct((128, 144), jnp.float32)],
    )(y, stats, pmat, pt, p["ff"]["Ws"][0], p["ff"]["bs"][0][None],
      p["ff"]["Ws"][1], p["ff"]["bs"][1][None])


def _t4_body(y2_ref, stats_ref, pb_ref, o_ref):
    mi = jnp.dot(pb_ref[...], _stats_to_mi(stats_ref[...]),
                 preferred_element_type=jnp.float32)
    o_ref[...] = (y2_ref[...] - mi[:, 0:1]) * mi[:, 1:2]


def _t4(y2, stats2, pmat):
    return pl.pallas_call(
        _t4_body,
        grid=(NPAD // BM,),
        in_specs=[_rows(64), _full((128, 144)), _rows(128)],
        out_specs=_rows(64),
        out_shape=jax.ShapeDtypeStruct((NPAD, 64), jnp.float32),
    )(y2, stats2, pmat)


def _tfa_body(o_ref, cr_ref, pt_ref, m_ref):
    i = pl.program_id(0)
    yy = jnp.concatenate([o_ref[...], cr_ref[...]], axis=1)
    st = jnp.dot(pt_ref[...], yy, preferred_element_type=jnp.float32)

    @pl.when(i == 0)
    def _():
        m_ref[...] = jnp.zeros_like(m_ref)

    m_ref[...] += st


def _tfa(o, colreal, pt):
    return pl.pallas_call(
        _tfa_body,
        grid=(NPAD // BM,),
        in_specs=[_rows(64), _rows(16), pl.BlockSpec((128, BM), lambda i: (0, i))],
        out_specs=_full((128, 80)),
        out_shape=jax.ShapeDtypeStruct((128, 80), jnp.float32),
    )(o, colreal, pt)


def _tfb_body(m_ref, ov_ref, c_ref, glob_ref):
    m = m_ref[...]
    ov = ov_ref[...]
    gmp = (m[:, 0:64] - ov) / jnp.maximum(m[:, 64:65], 1.0)
    glob_ref[...] = jnp.concatenate([gmp, ov, c_ref[...]], axis=1)


def _tfb(m, o_virt, c):
    return pl.pallas_call(
        _tfb_body,
        grid=(1,),
        in_specs=[_full((128, 80)), _full((128, 64)), _full((128, 64))],
        out_specs=_full((128, 192)),
        out_shape=jax.ShapeDtypeStruct((128, 192), jnp.float32),
    )(m, o_virt, c)


def _tfc_body(o_ref, pb_ref, c_ref, out_ref):
    cb = jnp.dot(pb_ref[...], c_ref[...], preferred_element_type=jnp.float32)
    out_ref[...] = jnp.concatenate([o_ref[...], cb], axis=1)


def _tfc(o, pmat, c, n):
    bm = 1000
    return pl.pallas_call(
        _tfc_body,
        grid=(n // bm,),
        in_specs=[pl.BlockSpec((bm, 64), lambda i: (i, 0)),
                  pl.BlockSpec((bm, 128), lambda i: (i, 0)),
                  _full((128, 64))],
        out_specs=pl.BlockSpec((bm, 128), lambda i: (i, 0)),
        out_shape=jax.ShapeDtypeStruct((n, 128), jnp.float32),
    )(o, pmat, c)


def _ldiv_body(la_ref, out_ref):
    la = la_ref[...]
    out_ref[...] = la[:, 0:64] / jnp.maximum(la[:, 64:65], 1.0)


def _ldiv(la):
    return pl.pallas_call(
        _ldiv_body,
        grid=(NPAD // BM,),
        in_specs=[_rows(80)],
        out_specs=_rows(64),
        out_shape=jax.ShapeDtypeStruct((NPAD, 64), jnp.float32),
    )(la)


def kernel(x, edge_index, edge_attr, batch, cond, params):
    N, G = x.shape[0], cond.shape[0]
    o = _mlp3(x, params["x2h"])
    e = _mlp3(edge_attr, params["e2h"])
    c = _mlp3(cond, params["c2h"], bm=128)

    u = jnp.arange(N, dtype=edge_index.dtype)
    v = batch.astype(edge_index.dtype) + N
    n_total = N + G
    NE = edge_index.shape[1]
    o = jnp.concatenate([o, c,
                         jnp.zeros((NPAD - n_total, 64), jnp.float32)], axis=0)

    rn = jnp.arange(NPAD)
    batch_pad = jnp.concatenate([
        batch.astype(jnp.int32), jnp.arange(G, dtype=jnp.int32),
        jnp.zeros((NPAD - n_total,), jnp.int32)])
    pmat = ((batch_pad[:, None] == jnp.arange(G, dtype=jnp.int32)[None, :])
            & (rn[:, None] < n_total)).astype(jnp.float32)
    pt = pmat.T
    colreal = ((rn[:, None] < N)
               & (jnp.arange(16)[None, :] == 0)).astype(jnp.float32)

    def _sortset(srcs, dsts, eids, gran):
        E = dsts.shape[0]
        E_pad = 128 * pl.cdiv(E, 128) + EDGE_SLACK
        pad_e = E_pad - E
        perm = jnp.argsort(dsts)

        def _padi(a):
            return jnp.concatenate([a.astype(jnp.int32),
                                    jnp.zeros((pad_e,), jnp.int32)])

        dst_sorted = dsts[perm]
        nb = NPAD // gran
        offs = jnp.searchsorted(dst_sorted,
                                jnp.arange(nb + 1, dtype=jnp.int32) * gran)
        npadofs = 16 * pl.cdiv(nb + 16, 16)
        offs = jnp.concatenate([
            offs.astype(jnp.int32),
            jnp.full((npadofs - nb - 1,), E, jnp.int32),
        ])
        src_p = _padi(srcs[perm]) if srcs is not None else None
        return src_p, _padi(dst_sorted), _padi(eids[perm]), offs

    src0 = jnp.concatenate([edge_index[0], u, v])
    dst0 = jnp.concatenate([edge_index[1], v, u])
    eid0 = jnp.concatenate([jnp.arange(NE, dtype=jnp.int32),
                            jnp.full((2 * N,), NE, jnp.int32)])
    _, dst0_s, eid0_s, offs0 = _sortset(None, dst0, eid0, ROWS_W)

    e_p_row = jnp.zeros((1, 64), jnp.float32).at[0, 0].set(1.0)
    e_store0 = jnp.concatenate([e, e_p_row,
                                jnp.zeros((n_total + 7, 64), jnp.float32)])
    la = _sc_loop_attr(e_store0, dst0_s, eid0_s, offs0)
    loop_attr = _ldiv(la)
    e_store = lax.dynamic_update_slice(e_store0, loop_attr[:n_total], (NE + 1, 0))

    sl = jnp.arange(n_total, dtype=edge_index.dtype)
    aug_src = jnp.concatenate([src0, sl])
    aug_dst = jnp.concatenate([dst0, sl])
    eid = jnp.concatenate([eid0, NE + 1 + jnp.arange(n_total, dtype=jnp.int32)])
    src_s, dst_s, eid_s, offs = _sortset(aug_src, aug_dst, eid, ROWS_W)
    _, _, _, offs64 = _sortset(None, aug_dst, eid, HALF_W)

    for p in params["layers"]:
        agg = _sc_gen_msg(o, e_store, src_s, dst_s, eid_s, offs)
        dtab, stab, skip = _t1(o, agg, loop_attr, p)
        accA = _sc_attn(dtab, stab, e_store, src_s, dst_s, eid_s, offs64)
        y, stats = _t2(accA, skip, o, pt, p)
        y2, stats2 = _t3(y, stats, pmat, pt, p)
        o = _t4(y2, stats2, pmat)

    m = _tfa(o, colreal, pt)
    o_virt = o[N:N + G]
    glob = _tfb(m, o_virt, c)
    o_final = _tfc(o, pmat, c, N)
    return (o_final, glob)
